# baseline (speedup 1.0000x reference)
import jax
import jax.numpy as jnp
from jax import lax
from jax.experimental import pallas as pl
from jax.experimental.pallas import tpu as pltpu, tpu_sc as plsc

B, S, D = 4, 8192, 768
NC, NS = 2, 16
NW = NC * NS
ROWS_PER_W = S // NW
CH = 32
NCHUNK = ROWS_PER_W // CH
LANES = 16
NCOL = D // LANES
NSTEP = NCHUNK * B


NXBUF = 3


def _sc_body(x_hbm, pos_hbm, out_hbm, x_v0, x_v1, x_v2, pos_v0, pos_v1,
             ld0, ld1, ld2, st0, st1, st2, ps0, ps1):
    x_bufs = [x_v0, x_v1, x_v2]
    pos_bufs = [pos_v0, pos_v1]
    ld_sems = [ld0, ld1, ld2]
    st_sems = [st0, st1, st2]
    pos_sems = [ps0, ps1]

    wid = lax.axis_index("s") * NC + lax.axis_index("c")
    seq_row0 = wid * ROWS_PER_W

    def x_row(step):
        c, b = divmod(step, B)
        return b * S + seq_row0 + c * CH

    def start_x_load(step):
        return pltpu.async_copy(
            x_hbm.at[pl.ds(x_row(step), CH)], x_bufs[step % NXBUF],
            ld_sems[step % NXBUF])

    def start_pos_load(c):
        return pltpu.async_copy(
            pos_hbm.at[pl.ds(seq_row0 + c * CH, CH)], pos_bufs[c % 2],
            pos_sems[c % 2])

    ld_h = [None] * NSTEP
    st_h = [None] * NSTEP
    pos_h = [None] * NCHUNK

    pos_h[0] = start_pos_load(0)
    ld_h[0] = start_x_load(0)
    ld_h[1] = start_x_load(1)
    if NCHUNK > 1:
        pos_h[1] = start_pos_load(1)

    for s in range(NSTEP):
        c, b = divmod(s, B)
        if s + 2 < NSTEP:
            if s + 2 - NXBUF >= 0:
                st_h[s + 2 - NXBUF].wait()
            ld_h[s + 2] = start_x_load(s + 2)
        ld_h[s].wait()
        if b == 0:
            pos_h[c].wait()

        buf = x_bufs[s % NXBUF]
        pbuf = pos_bufs[c % 2]

        @plsc.parallel_loop(0, CH, 1, unroll=1)
        def _(r):
            @plsc.parallel_loop(0, D, LANES, unroll=8)
            def _inner(k):
                buf[r, pl.ds(k, LANES)] = (
                    buf[r, pl.ds(k, LANES)] + pbuf[r, pl.ds(k, LANES)]
                )

        st_h[s] = pltpu.async_copy(
            buf, out_hbm.at[pl.ds(x_row(s), CH)], st_sems[s % NXBUF])

        if b == B - 1 and c + 2 < NCHUNK:
            pos_h[c + 2] = start_pos_load(c + 2)

    for s in range(NSTEP - NXBUF, NSTEP):
        if s >= 0:
            st_h[s].wait()


@jax.jit
def kernel(x, pos_table):
    mesh = plsc.VectorSubcoreMesh(
        core_axis_name="c", subcore_axis_name="s", num_cores=NC, num_subcores=NS
    )
    sc_call = pl.kernel(
        _sc_body,
        out_type=jax.ShapeDtypeStruct((B * S, D), jnp.float32),
        mesh=mesh,
        scratch_types=(
            [pltpu.VMEM((CH, D), jnp.float32)] * (NXBUF + 2)
            + [pltpu.SemaphoreType.DMA] * (2 * NXBUF + 2)
        ),
    )
    out = sc_call(x.reshape(B * S, D), pos_table)
    return out.reshape(B, S, D)


# speedup vs baseline: 1.0496x; 1.0496x over previous
import jax
import jax.numpy as jnp
from jax import lax
from jax.experimental import pallas as pl
from jax.experimental.pallas import tpu as pltpu, tpu_sc as plsc

B, S, D = 4, 8192, 768
NC, NS = 2, 16
NW = NC * NS
ROWS_PER_W = S // NW
CH = 32
NCHUNK = ROWS_PER_W // CH
LANES = 16
NCOL = D // LANES
NSTEP = NCHUNK * B


NXBUF = 3


def _sc_body(x_hbm, pos_hbm, out_hbm, x_v0, x_v1, x_v2, pos_v0, pos_v1,
             ld0, ld1, ld2, st0, st1, st2, ps0, ps1):
    x_bufs = [x_v0, x_v1, x_v2]
    pos_bufs = [pos_v0, pos_v1]
    ld_sems = [ld0, ld1, ld2]
    st_sems = [st0, st1, st2]
    pos_sems = [ps0, ps1]

    wid = lax.axis_index("s") * NC + lax.axis_index("c")
    seq_row0 = wid * ROWS_PER_W

    def x_row(step):
        c, b = divmod(step, B)
        return b * S + seq_row0 + c * CH

    def start_x_load(step):
        return pltpu.async_copy(
            x_hbm.at[pl.ds(x_row(step), CH)], x_bufs[step % NXBUF],
            ld_sems[step % NXBUF])

    def start_pos_load(c):
        return pltpu.async_copy(
            pos_hbm.at[pl.ds(seq_row0 + c * CH, CH)], pos_bufs[c % 2],
            pos_sems[c % 2])

    ld_h = [None] * NSTEP
    st_h = [None] * NSTEP
    pos_h = [None] * NCHUNK

    pos_h[0] = start_pos_load(0)
    ld_h[0] = start_x_load(0)
    ld_h[1] = start_x_load(1)
    if NCHUNK > 1:
        pos_h[1] = start_pos_load(1)

    for s in range(NSTEP):
        c, b = divmod(s, B)
        if s + 2 < NSTEP:
            if s + 2 - NXBUF >= 0:
                st_h[s + 2 - NXBUF].wait()
            ld_h[s + 2] = start_x_load(s + 2)
        ld_h[s].wait()
        if b == 0:
            pos_h[c].wait()

        buf = x_bufs[s % NXBUF]
        pbuf = pos_bufs[c % 2]

        @plsc.parallel_loop(0, CH * NCOL, 1, unroll=16)
        def _(i):
            r = i // NCOL
            k = (i - r * NCOL) * LANES
            buf[r, pl.ds(k, LANES)] = (
                buf[r, pl.ds(k, LANES)] + pbuf[r, pl.ds(k, LANES)]
            )

        st_h[s] = pltpu.async_copy(
            buf, out_hbm.at[pl.ds(x_row(s), CH)], st_sems[s % NXBUF])

        if b == B - 1 and c + 2 < NCHUNK:
            pos_h[c + 2] = start_pos_load(c + 2)

    for s in range(NSTEP - NXBUF, NSTEP):
        if s >= 0:
            st_h[s].wait()


@jax.jit
def kernel(x, pos_table):
    mesh = plsc.VectorSubcoreMesh(
        core_axis_name="c", subcore_axis_name="s", num_cores=NC, num_subcores=NS
    )
    sc_call = pl.kernel(
        _sc_body,
        out_type=jax.ShapeDtypeStruct((B * S, D), jnp.float32),
        mesh=mesh,
        scratch_types=(
            [pltpu.VMEM((CH, D), jnp.float32)] * (NXBUF + 2)
            + [pltpu.SemaphoreType.DMA] * (2 * NXBUF + 2)
        ),
    )
    out = sc_call(x.reshape(B * S, D), pos_table)
    return out.reshape(B, S, D)


# speedup vs baseline: 1.1861x; 1.1301x over previous
import jax
import jax.numpy as jnp
from jax import lax
from jax.experimental import pallas as pl
from jax.experimental.pallas import tpu as pltpu, tpu_sc as plsc

B, S, D = 4, 8192, 768
NC, NS = 2, 16
NW = NC * NS
ROWS_PER_W = S // NW
CH = 32
NCHUNK = ROWS_PER_W // CH
LANES = 16
NCOL = D // LANES
NSTEP = NCHUNK * B
NXBUF = 3


def _sc_body(x_hbm, pos_hbm, out_hbm, x_v0, x_v1, x_v2, pos_v0, pos_v1,
             iv0, iv1, iv2, g0, g1, g2, st0, st1, st2, ps0, ps1):
    x_bufs = [x_v0, x_v1, x_v2]
    pos_bufs = [pos_v0, pos_v1]
    idx_bufs = [iv0, iv1, iv2]
    g_sems = [g0, g1, g2]
    st_sems = [st0, st1, st2]
    pos_sems = [ps0, ps1]

    wid = lax.axis_index("s") * NC + lax.axis_index("c")
    seq_row0 = wid * ROWS_PER_W
    iota = lax.iota(jnp.int32, LANES)

    def start_pos_load(c):
        return pltpu.async_copy(
            pos_hbm.at[pl.ds(seq_row0 + c * CH, CH)], pos_bufs[c % 2],
            pos_sems[c % 2])

    g_h = [None] * NSTEP
    st_h = [None] * NSTEP
    pos_h = [None] * NCHUNK

    pos_h[0] = start_pos_load(0)
    if NCHUNK > 1:
        pos_h[1] = start_pos_load(1)

    def prep(t):
        c_t, b_t = divmod(t, B)
        if t - NXBUF >= 0:
            st_h[t - NXBUF].wait()
        if b_t == 0:
            pos_h[c_t].wait()
        buf = x_bufs[t % NXBUF]
        pbuf = pos_bufs[c_t % 2]

        @plsc.parallel_loop(0, CH * NCOL, 1, unroll=8)
        def _(i):
            r = i // NCOL
            k = (i - r * NCOL) * LANES
            buf[r, pl.ds(k, LANES)] = pbuf[r, pl.ds(k, LANES)]

        iv = idx_bufs[t % NXBUF]
        base = b_t * S + seq_row0 + c_t * CH
        for v in range(CH // LANES):
            iv[pl.ds(v * LANES, LANES)] = base + v * LANES + iota
        g_h[t] = pltpu.async_copy(x_hbm.at[iv], buf, g_sems[t % NXBUF],
                                  add=True)
        if b_t == B - 1 and c_t + 2 < NCHUNK:
            pos_h[c_t + 2] = start_pos_load(c_t + 2)

    prep(0)
    prep(1)
    for s in range(NSTEP):
        if s + 2 < NSTEP:
            prep(s + 2)
        g_h[s].wait()
        c_s, b_s = divmod(s, B)
        st_h[s] = pltpu.async_copy(
            x_bufs[s % NXBUF],
            out_hbm.at[pl.ds(b_s * S + seq_row0 + c_s * CH, CH)],
            st_sems[s % NXBUF])

    for s in range(NSTEP - NXBUF, NSTEP):
        st_h[s].wait()


@jax.jit
def kernel(x, pos_table):
    mesh = plsc.VectorSubcoreMesh(
        core_axis_name="c", subcore_axis_name="s", num_cores=NC, num_subcores=NS
    )
    sc_call = pl.kernel(
        _sc_body,
        out_type=jax.ShapeDtypeStruct((B * S, D), jnp.float32),
        mesh=mesh,
        scratch_types=(
            [pltpu.VMEM((CH, D), jnp.float32)] * NXBUF
            + [pltpu.VMEM((CH, D), jnp.float32)] * 2
            + [pltpu.VMEM((CH,), jnp.int32)] * NXBUF
            + [pltpu.SemaphoreType.DMA] * (2 * NXBUF + 2)
        ),
    )
    out = sc_call(x.reshape(B * S, D), pos_table)
    return out.reshape(B, S, D)
